# baseline (speedup 1.0000x reference)
import jax
import jax.numpy as jnp
from jax import lax
from jax.experimental import pallas as pl
from jax.experimental.pallas import tpu as pltpu
from jax.experimental.pallas import tpu_sc as plsc

N = 10000
E = 320000
F = 128
NC = 2
NS = 16
L = 16

CE = 64
CH = 8
SB = 20
NB = 4
ES = NC * NS * SB * CH * CE
N2 = 10240
RPT = N2 // NS
NDEG = 10240
ND2 = 10048


def _rsqrt_approx(d):
  i = lax.bitcast_convert_type(d, jnp.int32)
  i = jnp.int32(0x5F3759DF) - lax.shift_right_logical(i, 1)
  y = lax.bitcast_convert_type(i, jnp.float32)
  for _ in range(3):
    y = y * (1.5 - 0.5 * d * y * y)
  return y


def _sc_body(x_hbm, se_hbm, de_hbm, we_hbm, t_hbm, dinv_hbm,
             deg_sh, p_sh, s_v, d_v, w_v, dinv_v, rows_v,
             gsem, ssem):
  cidx = lax.axis_index("c")
  sidx = lax.axis_index("s")
  wid = cidx * NS + sidx

  def zz(i, _):
    dinv_v[pl.ds(i * L, L)] = jnp.zeros((L,), jnp.float32)
    return _
  lax.fori_loop(0, RPT // L, zz, None)

  def zb(r, _):
    for k in range(F // L):
      rows_v[0, r, pl.ds(k * L, L)] = jnp.zeros((L,), jnp.float32)
    return _
  lax.fori_loop(0, CE, zb, None)

  pltpu.sync_copy(dinv_v.at[pl.ds(0, RPT)], deg_sh.at[pl.ds(sidx * RPT, RPT)])
  for k in range(RPT // CE):
    pltpu.sync_copy(rows_v.at[0], p_sh.at[pl.ds(sidx * RPT + k * CE, CE)])
  plsc.subcore_barrier()

  def deg_batch(i, _):
    g = i // SB
    h = i % SB
    pltpu.sync_copy(se_hbm.at[g * NS + sidx, h], s_v)
    pltpu.sync_copy(we_hbm.at[g * NS + sidx, h], w_v)

    def deg_issue(j, _):
      pltpu.async_copy(w_v.at[j], deg_sh.at[s_v.at[j]], ssem, add=True)
      return _
    lax.fori_loop(0, CH, deg_issue, None)

    def deg_drain(j, _):
      pltpu.make_async_copy(w_v.at[0], deg_sh.at[s_v.at[0]], ssem).wait()
      return _
    lax.fori_loop(0, CH, deg_drain, None)
    return _
  lax.fori_loop(0, NC * SB, deg_batch, None)
  plsc.subcore_barrier()

  pltpu.sync_copy(deg_sh.at[pl.ds(0, ND2)], dinv_v)

  def mk_dinv(i, _):
    d = dinv_v[pl.ds(i * L, L)]
    y = _rsqrt_approx(d)
    dinv_v[pl.ds(i * L, L)] = jnp.where(d > 0.0, y, 0.0)
    return _
  lax.fori_loop(0, ND2 // L, mk_dinv, None)

  @pl.when(sidx == 0)
  def _():
    pltpu.sync_copy(dinv_v, dinv_hbm.at[pl.ds(cidx * ND2, ND2)])

  def gather(j, buf):
    pltpu.async_copy(x_hbm.at[s_v.at[j]], rows_v.at[buf], gsem)

  def wait_gather(buf):
    pltpu.make_async_copy(x_hbm.at[s_v.at[0]], rows_v.at[buf], gsem).wait()

  def scatter(j, buf):
    pltpu.async_copy(rows_v.at[buf], p_sh.at[d_v.at[j]], ssem, add=True)

  def drain_scatter():
    pltpu.make_async_copy(rows_v.at[0], p_sh.at[d_v.at[0]], ssem).wait()

  def chunk_compute(j, buf):
    def scale_group(g, _):
      idx16 = s_v[j, pl.ds(g * L, L)]
      dvv = plsc.load_gather(dinv_v, [idx16])
      sv16 = w_v[j, pl.ds(g * L, L)] * dvv
      for lane in range(L):
        e = g * L + lane
        se = sv16[lane]
        for k in range(F // L):
          rows_v[buf, e, pl.ds(k * L, L)] = rows_v[buf, e, pl.ds(k * L, L)] * se
      return _
    lax.fori_loop(0, CE // L, scale_group, None)

  def spmm_batch(h, _):
    pltpu.sync_copy(se_hbm.at[wid, h], s_v)
    pltpu.sync_copy(de_hbm.at[wid, h], d_v)
    pltpu.sync_copy(we_hbm.at[wid, h], w_v)

    for p in range(NB - 1):
      gather(p, p)

    def spmm_step(jj, _):
      for b in range(NB):
        j = jj * NB + b

        @pl.when(j >= 1)
        def _():
          drain_scatter()

        @pl.when(j + NB - 1 < CH)
        def _():
          gather(j + NB - 1, (j + NB - 1) % NB)

        wait_gather(b)
        chunk_compute(j, b)
        scatter(j, b)
      return _
    lax.fori_loop(0, CH // NB, spmm_step, None)
    drain_scatter()
    return _
  lax.fori_loop(0, SB, spmm_batch, None)
  plsc.subcore_barrier()

  for k in range(RPT // CE):
    row0 = sidx * RPT + k * CE
    pltpu.sync_copy(p_sh.at[pl.ds(row0, CE)], t_hbm.at[cidx, pl.ds(row0, CE)])


_sc_spmm = pl.kernel(
    _sc_body,
    out_type=(jax.ShapeDtypeStruct((NC, N2, F), jnp.float32),
              jax.ShapeDtypeStruct((NC * ND2,), jnp.float32)),
    mesh=plsc.VectorSubcoreMesh(core_axis_name="c", subcore_axis_name="s"),
    compiler_params=pltpu.CompilerParams(needs_layout_passes=False),
    scratch_types=[
        pltpu.VMEM_SHARED((NDEG,), jnp.float32),
        pltpu.VMEM_SHARED((N2, F), jnp.float32),
        pltpu.VMEM((CH, CE), jnp.int32),
        pltpu.VMEM((CH, CE), jnp.int32),
        pltpu.VMEM((CH, CE), jnp.float32),
        pltpu.VMEM((ND2,), jnp.float32),
        pltpu.VMEM((NB, CE, F), jnp.float32),
        pltpu.SemaphoreType.DMA,
        pltpu.SemaphoreType.DMA,
    ],
)


def _tc_body(x_ref, t2_ref, dv_ref, wx_ref, wt_ref, b_ref, lw_ref, lb_ref, o_ref):
  t = (t2_ref[0] + t2_ref[1]) * (-dv_ref[...])
  a = jnp.dot(x_ref[...], wx_ref[...], preferred_element_type=jnp.float32)
  a = a + jnp.dot(t, wt_ref[...], preferred_element_type=jnp.float32)
  a = a + b_ref[...]
  z = jax.nn.sigmoid(a[:, :F])
  ht = jnp.tanh(a[:, F:])
  h = jax.nn.relu((1.0 - z) * ht)
  o_ref[...] = jnp.dot(h, lw_ref[...], preferred_element_type=jnp.float32) + lb_ref[0, 0]


def _pad_edges(a, n, fill):
  return jnp.concatenate([a, jnp.full((n - E,), fill, a.dtype)])


def kernel(x, edge_index, edge_weight, W_xz, b_xz, W_xr, b_xr, W_xh, b_xh,
           W_hz, b_hz, W_hr, b_hr, W_hh, b_hh, lin_w, lin_b):
  src = edge_index[0]
  dst = edge_index[1]
  zi = jnp.int32(0)
  zf = jnp.float32(0.0)
  se = _pad_edges(src, ES, zi).reshape(NC * NS, SB, CH, CE)
  de = _pad_edges(dst, ES, zi).reshape(NC * NS, SB, CH, CE)
  we = _pad_edges(edge_weight, ES, zf).reshape(NC * NS, SB, CH, CE)

  t2, dinv2 = _sc_spmm(x, se, de, we)
  dv = dinv2[:N].reshape(N, 1)

  wx = jnp.concatenate([W_xz[0], W_xh[0]], axis=1)
  wt = jnp.concatenate([W_xz[1], W_xh[1]], axis=1)
  bc = jnp.concatenate([b_xz + b_hz, b_xh + b_hh]).reshape(1, 2 * F)

  BR = 2000
  out = pl.pallas_call(
      _tc_body,
      grid=(N // BR,),
      in_specs=[
          pl.BlockSpec((BR, F), lambda i: (i, 0)),
          pl.BlockSpec((NC, BR, F), lambda i: (0, i, 0)),
          pl.BlockSpec((BR, 1), lambda i: (i, 0)),
          pl.BlockSpec((F, 2 * F), lambda i: (0, 0)),
          pl.BlockSpec((F, 2 * F), lambda i: (0, 0)),
          pl.BlockSpec((1, 2 * F), lambda i: (0, 0)),
          pl.BlockSpec((F, 1), lambda i: (0, 0)),
          pl.BlockSpec((1, 1), lambda i: (0, 0)),
      ],
      out_specs=pl.BlockSpec((BR, 1), lambda i: (i, 0)),
      out_shape=jax.ShapeDtypeStruct((N, 1), jnp.float32),
  )(x, t2, dv, wx, wt, bc, lin_w, lin_b.reshape(1, 1))
  return out


# speedup vs baseline: 1.0650x; 1.0650x over previous
import jax
import jax.numpy as jnp
from jax import lax
from jax.experimental import pallas as pl
from jax.experimental.pallas import tpu as pltpu
from jax.experimental.pallas import tpu_sc as plsc

N = 10000
E = 320000
F = 128
NC = 2
NS = 16
L = 16

CE = 128
CH = 8
SB = 10
ES = NC * NS * SB * CH * CE
N2 = 10240
RPT = N2 // NS
NDEG = 10240
ND2 = 10048


def _rsqrt_approx(d):
  i = lax.bitcast_convert_type(d, jnp.int32)
  i = jnp.int32(0x5F3759DF) - lax.shift_right_logical(i, 1)
  y = lax.bitcast_convert_type(i, jnp.float32)
  for _ in range(3):
    y = y * (1.5 - 0.5 * d * y * y)
  return y


def _sc_body(x_hbm, se_hbm, de_hbm, we_hbm, t_hbm, dinv_hbm,
             deg_sh, p_sh, s_v, d_v, w_v, dinv_v, rows_v,
             gsem, ssem):
  cidx = lax.axis_index("c")
  sidx = lax.axis_index("s")
  wid = cidx * NS + sidx

  def zz(i, _):
    dinv_v[pl.ds(i * L, L)] = jnp.zeros((L,), jnp.float32)
    return _
  lax.fori_loop(0, RPT // L, zz, None)

  def zb(r, _):
    for k in range(F // L):
      rows_v[0, r, pl.ds(k * L, L)] = jnp.zeros((L,), jnp.float32)
    return _
  lax.fori_loop(0, CE, zb, None)

  pltpu.sync_copy(dinv_v.at[pl.ds(0, RPT)], deg_sh.at[pl.ds(sidx * RPT, RPT)])
  for k in range(RPT // CE):
    pltpu.sync_copy(rows_v.at[0], p_sh.at[pl.ds(sidx * RPT + k * CE, CE)])
  plsc.subcore_barrier()

  for g in range(NC):
    for h in range(SB):
      pltpu.sync_copy(se_hbm.at[g * NS + sidx, h], s_v)
      pltpu.sync_copy(we_hbm.at[g * NS + sidx, h], w_v)

      def deg_issue(j, _):
        pltpu.async_copy(w_v.at[j], deg_sh.at[s_v.at[j]], ssem, add=True)
        return _
      lax.fori_loop(0, CH, deg_issue, None)

      def deg_drain(j, _):
        pltpu.make_async_copy(w_v.at[0], deg_sh.at[s_v.at[0]], ssem).wait()
        return _
      lax.fori_loop(0, CH, deg_drain, None)
  plsc.subcore_barrier()

  pltpu.sync_copy(deg_sh.at[pl.ds(0, ND2)], dinv_v)

  def mk_dinv(i, _):
    d = dinv_v[pl.ds(i * L, L)]
    y = _rsqrt_approx(d)
    dinv_v[pl.ds(i * L, L)] = jnp.where(d > 0.0, y, 0.0)
    return _
  lax.fori_loop(0, ND2 // L, mk_dinv, None)

  @pl.when(sidx == 0)
  def _():
    pltpu.sync_copy(dinv_v, dinv_hbm.at[pl.ds(cidx * ND2, ND2)])

  def gather(j, buf):
    pltpu.async_copy(x_hbm.at[s_v.at[j]], rows_v.at[buf], gsem)

  def wait_gather(buf):
    pltpu.make_async_copy(x_hbm.at[s_v.at[0]], rows_v.at[buf], gsem).wait()

  def scatter(j, buf):
    pltpu.async_copy(rows_v.at[buf], p_sh.at[d_v.at[j]], ssem, add=True)

  def drain_scatter():
    pltpu.make_async_copy(rows_v.at[0], p_sh.at[d_v.at[0]], ssem).wait()

  def chunk_compute(j, buf):
    def scale_group(g, _):
      idx16 = s_v[j, pl.ds(g * L, L)]
      dvv = plsc.load_gather(dinv_v, [idx16])
      sv16 = w_v[j, pl.ds(g * L, L)] * dvv
      for lane in range(L):
        e = g * L + lane
        se = sv16[lane]
        for k in range(F // L):
          rows_v[buf, e, pl.ds(k * L, L)] = rows_v[buf, e, pl.ds(k * L, L)] * se
      return _
    lax.fori_loop(0, CE // L, scale_group, None)

  for h in range(SB):
    pltpu.sync_copy(se_hbm.at[wid, h], s_v)
    pltpu.sync_copy(de_hbm.at[wid, h], d_v)
    pltpu.sync_copy(we_hbm.at[wid, h], w_v)

    gather(0, 0)

    def spmm_step(jj, _):
      for b in range(2):
        j = jj * 2 + b
        nb = 1 - b

        @pl.when(j >= 1)
        def _():
          drain_scatter()

        @pl.when(j + 1 < CH)
        def _():
          gather(j + 1, nb)

        wait_gather(b)
        chunk_compute(j, b)
        scatter(j, b)
      return _
    lax.fori_loop(0, CH // 2, spmm_step, None)
    drain_scatter()
  plsc.subcore_barrier()

  for k in range(RPT // CE):
    row0 = sidx * RPT + k * CE
    pltpu.sync_copy(p_sh.at[pl.ds(row0, CE)], t_hbm.at[cidx, pl.ds(row0, CE)])


_sc_spmm = pl.kernel(
    _sc_body,
    out_type=(jax.ShapeDtypeStruct((NC, N2, F), jnp.float32),
              jax.ShapeDtypeStruct((NC * ND2,), jnp.float32)),
    mesh=plsc.VectorSubcoreMesh(core_axis_name="c", subcore_axis_name="s"),
    compiler_params=pltpu.CompilerParams(needs_layout_passes=False),
    scratch_types=[
        pltpu.VMEM_SHARED((NDEG,), jnp.float32),
        pltpu.VMEM_SHARED((N2, F), jnp.float32),
        pltpu.VMEM((CH, CE), jnp.int32),
        pltpu.VMEM((CH, CE), jnp.int32),
        pltpu.VMEM((CH, CE), jnp.float32),
        pltpu.VMEM((ND2,), jnp.float32),
        pltpu.VMEM((2, CE, F), jnp.float32),
        pltpu.SemaphoreType.DMA,
        pltpu.SemaphoreType.DMA,
    ],
)


def _tc_body(x_ref, t2_ref, dv_ref, wx_ref, wt_ref, b_ref, lw_ref, lb_ref, o_ref):
  t = (t2_ref[0] + t2_ref[1]) * (-dv_ref[...])
  a = jnp.dot(x_ref[...], wx_ref[...], preferred_element_type=jnp.float32)
  a = a + jnp.dot(t, wt_ref[...], preferred_element_type=jnp.float32)
  a = a + b_ref[...]
  z = jax.nn.sigmoid(a[:, :F])
  ht = jnp.tanh(a[:, F:])
  h = jax.nn.relu((1.0 - z) * ht)
  o_ref[...] = jnp.dot(h, lw_ref[...], preferred_element_type=jnp.float32) + lb_ref[0, 0]


def _pad_edges(a, n, fill):
  return jnp.concatenate([a, jnp.full((n - E,), fill, a.dtype)])


def kernel(x, edge_index, edge_weight, W_xz, b_xz, W_xr, b_xr, W_xh, b_xh,
           W_hz, b_hz, W_hr, b_hr, W_hh, b_hh, lin_w, lin_b):
  src = edge_index[0]
  dst = edge_index[1]
  zi = jnp.int32(0)
  zf = jnp.float32(0.0)
  se = _pad_edges(src, ES, zi).reshape(NC * NS, SB, CH, CE)
  de = _pad_edges(dst, ES, zi).reshape(NC * NS, SB, CH, CE)
  we = _pad_edges(edge_weight, ES, zf).reshape(NC * NS, SB, CH, CE)

  t2, dinv2 = _sc_spmm(x, se, de, we)
  dv = dinv2[:N].reshape(N, 1)

  wx = jnp.concatenate([W_xz[0], W_xh[0]], axis=1)
  wt = jnp.concatenate([W_xz[1], W_xh[1]], axis=1)
  bc = jnp.concatenate([b_xz + b_hz, b_xh + b_hh]).reshape(1, 2 * F)

  BR = 2000
  out = pl.pallas_call(
      _tc_body,
      grid=(N // BR,),
      in_specs=[
          pl.BlockSpec((BR, F), lambda i: (i, 0)),
          pl.BlockSpec((NC, BR, F), lambda i: (0, i, 0)),
          pl.BlockSpec((BR, 1), lambda i: (i, 0)),
          pl.BlockSpec((F, 2 * F), lambda i: (0, 0)),
          pl.BlockSpec((F, 2 * F), lambda i: (0, 0)),
          pl.BlockSpec((1, 2 * F), lambda i: (0, 0)),
          pl.BlockSpec((F, 1), lambda i: (0, 0)),
          pl.BlockSpec((1, 1), lambda i: (0, 0)),
      ],
      out_specs=pl.BlockSpec((BR, 1), lambda i: (i, 0)),
      out_shape=jax.ShapeDtypeStruct((N, 1), jnp.float32),
  )(x, t2, dv, wx, wt, bc, lin_w, lin_b.reshape(1, 1))
  return out
